# Initial kernel scaffold; baseline (speedup 1.0000x reference)
import jax
import jax.numpy as jnp
from jax.experimental import pallas as pl


def kernel(y, edge_index):
    raise NotImplementedError("write your pallas kernel here")


# speedup vs baseline: 247.0773x; 247.0773x over previous
import functools

import jax
import jax.numpy as jnp
from jax import lax
from jax.experimental import pallas as pl
from jax.experimental.pallas import tpu as pltpu
from jax.experimental.pallas import tpu_sc as plsc

N_NODES = 100000
N_EDGES = 6400000
INV_TEMP = 10.0
NEG_INF = 1000000.0
STEPS = 10

NUM_TILES = 16
NODES_PAD = 102400
NODES_PER_TILE = NODES_PAD // NUM_TILES
EDGES_PER_TILE = N_EDGES // NUM_TILES
CHUNK = 16000
N_CHUNKS = EDGES_PER_TILE // CHUNK
U_INIT = 32.0

_LN2_HI = 0.693359375
_LN2_LO = -2.12194440e-4


def _log16(x):
    x = jnp.maximum(x, 1e-37)
    xi = lax.bitcast_convert_type(x, jnp.int32)
    e = ((xi >> 23) - 127).astype(jnp.float32)
    m = lax.bitcast_convert_type((xi & 0x007FFFFF) | 0x3F800000, jnp.float32)
    big = m > 1.41421356
    m = jnp.where(big, m * 0.5, m)
    e = jnp.where(big, e + 1.0, e)
    t = m - 1.0
    z = t * t
    p = jnp.full((16,), 7.0376836292e-2, jnp.float32)
    p = p * t + (-1.1514610310e-1)
    p = p * t + 1.1676998740e-1
    p = p * t + (-1.2420140846e-1)
    p = p * t + 1.4249322787e-1
    p = p * t + (-1.6668057665e-1)
    p = p * t + 2.0000714765e-1
    p = p * t + (-2.4999993993e-1)
    p = p * t + 3.3333331174e-1
    y = t * z * p - 0.5 * z
    r = e * _LN2_LO + y
    r = r + t
    return r + e * _LN2_HI


def _mesh():
    return plsc.VectorSubcoreMesh(
        core_axis_name="c", subcore_axis_name="s", num_cores=1,
        num_subcores=NUM_TILES)


@functools.partial(
    pl.kernel,
    out_type=jax.ShapeDtypeStruct((N_EDGES,), jnp.float32),
    mesh=_mesh(),
    scratch_types=[
        pltpu.VMEM_SHARED((NODES_PAD,), jnp.float32),
        pltpu.VMEM_SHARED((NODES_PAD,), jnp.float32),
        pltpu.VMEM_SHARED((NODES_PAD,), jnp.float32),
        pltpu.VMEM((CHUNK,), jnp.float32),
        pltpu.VMEM((CHUNK,), jnp.int32),
        pltpu.VMEM((CHUNK,), jnp.int32),
        pltpu.VMEM((CHUNK,), jnp.float32),
        pltpu.VMEM((CHUNK,), jnp.float32),
        pltpu.VMEM((CHUNK,), jnp.float32),
        pltpu.VMEM((NODES_PER_TILE,), jnp.float32),
        pltpu.VMEM((NODES_PER_TILE,), jnp.float32),
        pltpu.SemaphoreType.DMA,
    ],
)
def _sinkhorn(y_hbm, f_hbm, t_hbm, out_hbm,
              u, v, acc, y_b, f_b, t_b, uf_b, vt_b, p_b, a_b, n_b, sem):
    wid = lax.axis_index("s")
    nsl = pl.ds(wid * NODES_PER_TILE, NODES_PER_TILE)

    @pl.loop(0, NODES_PER_TILE // 16)
    def _(i):
        sl = pl.ds(i * 16, 16)
        a_b[sl] = jnp.zeros((16,), jnp.float32)
        n_b[sl] = jnp.full((16,), U_INIT, jnp.float32)

    pltpu.sync_copy(n_b, u.at[nsl])
    pltpu.sync_copy(a_b, v.at[nsl])
    pltpu.sync_copy(a_b, acc.at[nsl])
    plsc.subcore_barrier()

    def edge_pass(scatter: bool, is_row=None):
        @pl.loop(0, N_CHUNKS)
        def _(ci):
            base = wid * EDGES_PER_TILE + ci * CHUNK
            esl = pl.ds(base, CHUNK)
            pltpu.sync_copy(y_hbm.at[esl], y_b)
            pltpu.sync_copy(f_hbm.at[esl], f_b)
            pltpu.sync_copy(t_hbm.at[esl], t_b)
            pltpu.async_copy(u.at[f_b], uf_b, sem).wait()
            pltpu.async_copy(v.at[t_b], vt_b, sem).wait()

            @pl.loop(0, CHUNK // 16)
            def _(i):
                sl = pl.ds(i * 16, 16)
                fv = f_b[sl]
                tv = t_b[sl]
                zv = jnp.where(fv == tv, -NEG_INF, y_b[sl] * INV_TEMP)
                zv = zv - uf_b[sl] - vt_b[sl]
                p_b[sl] = jnp.exp(zv) if scatter else zv

            if scatter:
                @pl.when(is_row)
                def _():
                    pltpu.sync_copy(p_b, acc.at[f_b], add=True)

                @pl.when(jnp.logical_not(is_row))
                def _():
                    pltpu.sync_copy(p_b, acc.at[t_b], add=True)
            else:
                pltpu.sync_copy(p_b, out_hbm.at[esl])

    @pl.loop(0, 2 * STEPS)
    def _(step):
        is_row = (step & 1) == 0
        edge_pass(scatter=True, is_row=is_row)
        plsc.subcore_barrier()

        pltpu.sync_copy(acc.at[nsl], a_b)

        @pl.when(is_row)
        def _():
            pltpu.sync_copy(u.at[nsl], n_b)

        @pl.when(jnp.logical_not(is_row))
        def _():
            pltpu.sync_copy(v.at[nsl], n_b)

        @pl.loop(0, NODES_PER_TILE // 16)
        def _(i):
            sl = pl.ds(i * 16, 16)
            n_b[sl] = n_b[sl] + _log16(a_b[sl])
            a_b[sl] = jnp.zeros((16,), jnp.float32)

        @pl.when(is_row)
        def _():
            pltpu.sync_copy(n_b, u.at[nsl])

        @pl.when(jnp.logical_not(is_row))
        def _():
            pltpu.sync_copy(n_b, v.at[nsl])

        pltpu.sync_copy(a_b, acc.at[nsl])
        plsc.subcore_barrier()

    edge_pass(scatter=False)


def kernel(y, edge_index):
    return _sinkhorn(y, edge_index[0], edge_index[1])
